# baseline (speedup 1.0000x reference)
import functools

import jax
import jax.numpy as jnp
from jax import lax
from jax.experimental import pallas as pl
from jax.experimental.pallas import tpu as pltpu
from jax.experimental.pallas import tpu_sc as plsc

N = 10000
E = 320000
F = 128
NCLASS = 40
NGRAPHS = 16

NCORES = 2
NSUB = 16
NTILES = NCORES * NSUB
NPAD = 10240
ROWS_PER_TILE = NPAD // NSUB
C = 128
EPT = E // NTILES
CH = 80
EPT_PAD = CH * C
HN = NPAD // 2
QROWS = CH + 4
TROWS = HN + 8
AROWS = 5376
AZPT = AROWS // NSUB
HPT = HN // NSUB
DUMMY = HN
ZROW = N
RB = 1024
GRID = NPAD // RB


def _sc_degree_body(src_hbm, dst_hbm,
                    degp_hbm, srcq_hbm, dstq_hbm, rb_hbm,
                    src_v, dst_v, acc, sq, dq, rb_v):
    wid = lax.axis_index("s") * NCORES + lax.axis_index("c")
    pltpu.sync_copy(src_hbm.at[wid], src_v)
    pltpu.sync_copy(dst_hbm.at[wid], dst_v)
    zero = jnp.zeros((16,), jnp.float32)

    def zbody(i, c):
        acc[pl.ds(i * 16, 16)] = zero
        return c

    lax.fori_loop(0, NPAD // 16, zbody, 0)
    one = jnp.ones((16,), jnp.float32)
    dummy = jnp.full((16,), DUMMY, jnp.int32)

    def fbody(r, c):
        for k in range(C // 16):
            sq[r, pl.ds(k * 16, 16)] = dummy
            dq[r, pl.ds(k * 16, 16)] = dummy
        return c

    lax.fori_loop(0, QROWS, fbody, 0)

    def ebody(j, c):
        for k in range(C // 16):
            idx = dst_v[j, pl.ds(k * 16, 16)]
            plsc.addupdate_scatter(acc, [idx], one)
        return c

    lax.fori_loop(0, CH, ebody, 0)
    pltpu.sync_copy(acc, degp_hbm.at[wid])

    hn = jnp.full((16,), HN, jnp.int32)

    def cbody(j, cnts):
        c0, c1, c2, c3 = cnts
        for k in range(C // 16):
            sv = src_v[j, pl.ds(k * 16, 16)]
            dv = dst_v[j, pl.ds(k * 16, 16)]
            b_id = ((dv >= hn).astype(jnp.int32) * 2
                    + (sv >= hn).astype(jnp.int32))
            c0 += plsc.all_reduce_population_count(b_id == 0)
            c1 += plsc.all_reduce_population_count(b_id == 1)
            c2 += plsc.all_reduce_population_count(b_id == 2)
            c3 += plsc.all_reduce_population_count(b_id == 3)
        return c0, c1, c2, c3

    zi = jnp.zeros((16,), jnp.int32)
    c0, c1, c2, c3 = lax.fori_loop(0, CH, cbody, (zi, zi, zi, zi))

    r1 = (c0 + (C - 1)) >> 7
    r2 = r1 + ((c1 + (C - 1)) >> 7)
    r3 = r2 + ((c2 + (C - 1)) >> 7)
    r4 = r3 + ((c3 + (C - 1)) >> 7)
    rb_v[0] = zi
    rb_v[1] = r1
    rb_v[2] = r2
    rb_v[3] = r3
    rb_v[4] = r4
    pltpu.sync_copy(rb_v, rb_hbm.at[wid])

    e1 = r1 << 7
    e2 = r2 << 7
    e3 = r3 << 7

    def pbody(j, es):
        e0, e1, e2, e3 = es
        bases = [e0, e1, e2, e3]
        for k in range(C // 16):
            sv = src_v[j, pl.ds(k * 16, 16)]
            dv = dst_v[j, pl.ds(k * 16, 16)]
            shi = (sv >= hn)
            dhi = (dv >= hn)
            b_id = dhi.astype(jnp.int32) * 2 + shi.astype(jnp.int32)
            srel = sv - shi.astype(jnp.int32) * HN
            drel = dv - dhi.astype(jnp.int32) * HN
            for b in range(4):
                m = b_id == b
                pos = bases[b] + plsc.cumsum(m.astype(jnp.int32)) - 1
                row = pos >> 7
                col = pos & (C - 1)
                plsc.store_scatter(sq, [row, col], srel, mask=m)
                plsc.store_scatter(dq, [row, col], drel, mask=m)
                bases[b] = bases[b] + plsc.all_reduce_population_count(m)
        return tuple(bases)

    lax.fori_loop(0, CH, pbody, (zi, e1, e2, e3))
    pltpu.sync_copy(sq, srcq_hbm.at[wid])
    pltpu.sync_copy(dq, dstq_hbm.at[wid])


def _sc_aggregate_body(g_hbm, srcq_hbm, dstq_hbm, rb_hbm, out_hbm,
                       sq, dq, rb_v, rows_v, zbuf, table, acc):
    cid = lax.axis_index("c")
    sid = lax.axis_index("s")
    wid = sid * NCORES + cid
    pltpu.sync_copy(srcq_hbm.at[wid], sq)
    pltpu.sync_copy(dstq_hbm.at[wid], dq)
    pltpu.sync_copy(rb_hbm.at[wid], rb_v)
    zero = jnp.zeros((16,), jnp.float32)
    zrows = zbuf.shape[0]

    def zbody(r, c):
        for k in range(F // 16):
            zbuf[r, pl.ds(k * 16, 16)] = zero
        return c

    lax.fori_loop(0, zrows, zbody, 0)

    def zero_acc():
        base = sid * AZPT
        for b in range(AZPT // zrows):
            pltpu.sync_copy(zbuf, acc.at[pl.ds(base + b * zrows, zrows)])
        rem = AZPT % zrows
        if rem:
            pltpu.sync_copy(zbuf.at[pl.ds(0, rem)],
                            acc.at[pl.ds(base + AZPT - rem, rem)])

    def run_bucket(p):
        lo = rb_v[p][0]
        hi = rb_v[p + 1][0]

        def ebody(r, c):
            pltpu.sync_copy(table.at[sq.at[r]], rows_v)
            pltpu.sync_copy(rows_v, acc.at[dq.at[r]], add=True)
            return c

        lax.fori_loop(lo, hi, ebody, 0)

    for h in range(2):
        zero_acc()
        for sh in range(2):
            plsc.subcore_barrier()
            pltpu.sync_copy(g_hbm.at[pl.ds(sh * HN + sid * HPT, HPT)],
                            table.at[pl.ds(sid * HPT, HPT)])

            @pl.when(sid == 0)
            def _():
                pltpu.sync_copy(zbuf.at[pl.ds(0, 8)],
                                table.at[pl.ds(HN, 8)])

            plsc.subcore_barrier()
            run_bucket(h * 2 + sh)
        plsc.subcore_barrier()
        pltpu.sync_copy(acc.at[pl.ds(sid * HPT, HPT)],
                        out_hbm.at[cid, pl.ds(h * HN + sid * HPT, HPT)])
        plsc.subcore_barrier()


@functools.cache
def _build_sc_kernels():
    mesh = plsc.VectorSubcoreMesh(
        core_axis_name="c", subcore_axis_name="s",
        num_cores=NCORES, num_subcores=NSUB)
    sc_degree = pl.kernel(
        _sc_degree_body,
        out_type=(
            jax.ShapeDtypeStruct((NTILES, NPAD), jnp.float32),
            jax.ShapeDtypeStruct((NTILES, QROWS, C), jnp.int32),
            jax.ShapeDtypeStruct((NTILES, QROWS, C), jnp.int32),
            jax.ShapeDtypeStruct((NTILES, 5, 16), jnp.int32),
        ),
        mesh=mesh,
        compiler_params=pltpu.CompilerParams(needs_layout_passes=False),
        scratch_types=[
            pltpu.VMEM((CH, C), jnp.int32),
            pltpu.VMEM((CH, C), jnp.int32),
            pltpu.VMEM((NPAD,), jnp.float32),
            pltpu.VMEM((QROWS, C), jnp.int32),
            pltpu.VMEM((QROWS, C), jnp.int32),
            pltpu.VMEM((5, 16), jnp.int32),
        ],
    )
    sc_aggregate = pl.kernel(
        _sc_aggregate_body,
        out_type=jax.ShapeDtypeStruct((NCORES, NPAD, F), jnp.float32),
        mesh=mesh,
        scratch_types=[
            pltpu.VMEM((QROWS, C), jnp.int32),
            pltpu.VMEM((QROWS, C), jnp.int32),
            pltpu.VMEM((5, 16), jnp.int32),
            pltpu.VMEM((C, F), jnp.float32),
            pltpu.VMEM((32, F), jnp.float32),
            pltpu.VMEM_SHARED((TROWS, F), jnp.float32),
            pltpu.VMEM_SHARED((AROWS, F), jnp.float32),
        ],
    )
    return sc_degree, sc_aggregate


def _tc_prep_body(x_ref, w1_ref, degp_ref, h1_ref, g1_ref, dinv_ref, invd_ref):
    h1 = jnp.dot(x_ref[...], w1_ref[...], preferred_element_type=jnp.float32)
    deg = jnp.sum(degp_ref[...], axis=1, keepdims=True) + 1.0
    dinv = lax.rsqrt(deg)
    h1_ref[...] = h1
    g1_ref[...] = h1 * dinv
    dinv_ref[...] = dinv
    invd_ref[...] = 1.0 / deg


def _tc_layer2_body(a0_ref, a1_ref, h1_ref, dinv_ref, invd_ref, b1_ref, w2_ref,
                    h2_ref, g2_ref):
    dinv = dinv_ref[...]
    out1 = jnp.maximum(
        dinv * (a0_ref[...] + a1_ref[...]) + invd_ref[...] * h1_ref[...]
        + b1_ref[...], 0.0)
    h2 = jnp.dot(out1, w2_ref[...], preferred_element_type=jnp.float32)
    rid = pl.program_id(0) * RB + lax.broadcasted_iota(jnp.int32, (RB, 1), 0)
    valid = (rid < N).astype(jnp.float32)
    h2_ref[...] = h2
    g2_ref[...] = h2 * dinv * valid


def _tc_pool_body(a0_ref, a1_ref, h2_ref, dinv_ref, invd_ref, b2_ref,
                  batch_ref, sums_ref, cnt_ref):
    out2 = jnp.maximum(
        dinv_ref[...] * (a0_ref[...] + a1_ref[...])
        + invd_ref[...] * h2_ref[...] + b2_ref[...], 0.0)
    brow = batch_ref[0]
    giota = lax.broadcasted_iota(jnp.int32, (NGRAPHS, RB), 0)
    onehot_t = (giota == brow).astype(jnp.float32)
    psum = jnp.dot(onehot_t, out2, preferred_element_type=jnp.float32)
    pcnt = jnp.sum(onehot_t, axis=1, keepdims=True)

    @pl.when(pl.program_id(0) == 0)
    def _():
        sums_ref[...] = jnp.zeros_like(sums_ref)
        cnt_ref[...] = jnp.zeros_like(cnt_ref)

    sums_ref[...] += psum
    cnt_ref[...] += jnp.broadcast_to(pcnt, (NGRAPHS, F))


def _tc_head_body(sums_ref, cnt_ref, wlin_ref, blin_ref, out_ref):
    pooled = sums_ref[...] / jnp.maximum(cnt_ref[...], 1.0)
    logits = jnp.dot(pooled, wlin_ref[...],
                     preferred_element_type=jnp.float32) + blin_ref[...]
    m = jnp.max(logits, axis=1, keepdims=True)
    e = jnp.exp(logits - m)
    out_ref[...] = e / jnp.sum(e, axis=1, keepdims=True)


def _row(i):
    return (i, 0)


def _rep(i):
    return (0, 0)


_tc_prep = pl.pallas_call(
    _tc_prep_body,
    grid=(GRID,),
    in_specs=[
        pl.BlockSpec((RB, F), _row),
        pl.BlockSpec((F, F), _rep),
        pl.BlockSpec((RB, NTILES), _row),
    ],
    out_specs=[
        pl.BlockSpec((RB, F), _row),
        pl.BlockSpec((RB, F), _row),
        pl.BlockSpec((RB, 1), _row),
        pl.BlockSpec((RB, 1), _row),
    ],
    out_shape=[
        jax.ShapeDtypeStruct((NPAD, F), jnp.float32),
        jax.ShapeDtypeStruct((NPAD, F), jnp.float32),
        jax.ShapeDtypeStruct((NPAD, 1), jnp.float32),
        jax.ShapeDtypeStruct((NPAD, 1), jnp.float32),
    ],
)

_tc_layer2 = pl.pallas_call(
    _tc_layer2_body,
    grid=(GRID,),
    in_specs=[
        pl.BlockSpec((RB, F), _row),
        pl.BlockSpec((RB, F), _row),
        pl.BlockSpec((RB, F), _row),
        pl.BlockSpec((RB, 1), _row),
        pl.BlockSpec((RB, 1), _row),
        pl.BlockSpec((1, F), _rep),
        pl.BlockSpec((F, F), _rep),
    ],
    out_specs=[
        pl.BlockSpec((RB, F), _row),
        pl.BlockSpec((RB, F), _row),
    ],
    out_shape=[
        jax.ShapeDtypeStruct((NPAD, F), jnp.float32),
        jax.ShapeDtypeStruct((NPAD, F), jnp.float32),
    ],
)

_tc_pool = pl.pallas_call(
    _tc_pool_body,
    grid=(GRID,),
    in_specs=[
        pl.BlockSpec((RB, F), _row),
        pl.BlockSpec((RB, F), _row),
        pl.BlockSpec((RB, F), _row),
        pl.BlockSpec((RB, 1), _row),
        pl.BlockSpec((RB, 1), _row),
        pl.BlockSpec((1, F), _rep),
        pl.BlockSpec((1, 1, RB), lambda i: (i, 0, 0)),
    ],
    out_specs=[
        pl.BlockSpec((NGRAPHS, F), _rep),
        pl.BlockSpec((NGRAPHS, F), _rep),
    ],
    out_shape=[
        jax.ShapeDtypeStruct((NGRAPHS, F), jnp.float32),
        jax.ShapeDtypeStruct((NGRAPHS, F), jnp.float32),
    ],
)

_tc_head = pl.pallas_call(
    _tc_head_body,
    out_shape=jax.ShapeDtypeStruct((NGRAPHS, F), jnp.float32),
)


def kernel(x, edge_index, edge_attr, batch, W1, b1, W2, b2, Wlin, blin):
    x_pad = jnp.zeros((NPAD, F), jnp.float32).at[:N].set(x)
    src = edge_index[0].astype(jnp.int32).reshape(NTILES, EPT)
    dst = edge_index[1].astype(jnp.int32).reshape(NTILES, EPT)
    pad = ((0, 0), (0, EPT_PAD - EPT))
    src3 = jnp.pad(src, pad, constant_values=ZROW).reshape(NTILES, CH, C)
    dst3 = jnp.pad(dst, pad, constant_values=NPAD - 1).reshape(NTILES, CH, C)
    batch_rs = jnp.pad(batch.astype(jnp.int32), (0, NPAD - N),
                       constant_values=NGRAPHS).reshape(GRID, 1, RB)
    b1r = b1.reshape(1, F)
    b2r = b2.reshape(1, F)
    wlin_pad = jnp.zeros((F, F), jnp.float32).at[:, :NCLASS].set(Wlin)
    blin_row = jnp.full((1, F), -1e30, jnp.float32).at[0, :NCLASS].set(blin)

    _sc_degree, _sc_aggregate = _build_sc_kernels()
    degp, srcq, dstq, rb = _sc_degree(src3, dst3)
    h1, g1, dinv, invd = _tc_prep(x_pad, W1, degp.T)
    agg1 = _sc_aggregate(g1, srcq, dstq, rb)
    h2, g2 = _tc_layer2(agg1[0], agg1[1], h1, dinv, invd, b1r, W2)
    agg2 = _sc_aggregate(g2, srcq, dstq, rb)
    sums, cnt = _tc_pool(agg2[0], agg2[1], h2, dinv, invd, b2r, batch_rs)
    probs = _tc_head(sums, cnt, wlin_pad, blin_row)
    return probs[:, :NCLASS]
